# Initial kernel scaffold; baseline (speedup 1.0000x reference)
import functools

import jax
import jax.numpy as jnp
from jax import lax
from jax.experimental import pallas as pl
from jax.experimental.pallas import tpu as pltpu
from jax.experimental.pallas import tpu_sc as plsc

F32 = jnp.float32


def _edgek_body(edges_ref, w_ref, b_ref, out_ref):
    out_ref[...] = (
        jnp.dot(edges_ref[...], w_ref[...], preferred_element_type=F32)
        + b_ref[...]
    )


def _edge_keys(edges, wk2, bk2, block_e=8000):
    e = edges.shape[0]
    assert e % block_e == 0
    return pl.pallas_call(
        _edgek_body,
        grid=(e // block_e,),
        in_specs=[
            pl.BlockSpec((block_e, edges.shape[1]), lambda i: (i, 0)),
            pl.BlockSpec(wk2.shape, lambda i: (0, 0)),
            pl.BlockSpec((1, bk2.shape[1]), lambda i: (0, 0)),
        ],
        out_specs=pl.BlockSpec((block_e, 8), lambda i: (i, 0)),
        out_shape=jax.ShapeDtypeStruct((e, 8), F32),
    )(edges, wk2, bk2)


_NW = 32
_CHUNK = 125
_ROUND = 50


def _gather_rows(idx2d, table):
    nch = idx2d.shape[0]
    g = nch * _CHUNK
    cpt = nch // _NW
    nr = cpt // _ROUND
    assert cpt * _NW == nch and nr * _ROUND == cpt
    mesh = plsc.VectorSubcoreMesh(core_axis_name="c", subcore_axis_name="s")

    @functools.partial(
        pl.kernel,
        out_type=jax.ShapeDtypeStruct((g, 8), F32),
        mesh=mesh,
        scratch_types=[
            pltpu.VMEM((cpt, _CHUNK), jnp.int32),
            pltpu.VMEM((_ROUND * _CHUNK, 8), F32),
            pltpu.SemaphoreType.DMA,
        ],
    )
    def _gather(idx_hbm, table_hbm, out_hbm, idx_v, rows_v, gsem):
        c = lax.axis_index("c")
        s = lax.axis_index("s")
        w = s * 2 + c
        base = w * cpt
        pltpu.sync_copy(idx_hbm.at[pl.ds(base, cpt)], idx_v)

        def round_body(r, carry):
            for b in range(_ROUND):
                pltpu.async_copy(
                    table_hbm.at[idx_v.at[r * _ROUND + b]],
                    rows_v.at[pl.ds(b * _CHUNK, _CHUNK)],
                    gsem,
                )
            for b in range(_ROUND):
                pltpu.make_async_copy(
                    table_hbm.at[idx_v.at[r * _ROUND + b]],
                    rows_v.at[pl.ds(b * _CHUNK, _CHUNK)],
                    gsem,
                ).wait()
            pltpu.sync_copy(
                rows_v,
                out_hbm.at[pl.ds((base + r * _ROUND) * _CHUNK, _ROUND * _CHUNK)],
            )
            return carry

        lax.fori_loop(0, nr, round_body, 0)

    return _gather(idx2d, table)


def _segment_sum_parts(seg2d, feat, n):
    nch = seg2d.shape[0]
    d = feat.shape[1]
    cpt = nch // _NW
    nr = cpt // _ROUND
    assert cpt * _NW == nch and nr * _ROUND == cpt
    nps = n // 16
    assert nps * 16 == n and nps % _CHUNK == 0
    mesh = plsc.VectorSubcoreMesh(core_axis_name="c", subcore_axis_name="s")

    @functools.partial(
        pl.kernel,
        out_type=jax.ShapeDtypeStruct((2 * n, d), F32),
        mesh=mesh,
        scratch_types=[
            pltpu.VMEM((cpt, _CHUNK), jnp.int32),
            pltpu.VMEM((_ROUND * _CHUNK, d), F32),
            pltpu.VMEM((_CHUNK, d), F32),
            pltpu.VMEM_SHARED((n, d), F32),
            pltpu.SemaphoreType.DMA,
        ],
    )
    def _scatter(seg_hbm, feat_hbm, out_hbm, idx_v, rows_v, zero_v, acc_sh, ssem):
        c = lax.axis_index("c")
        s = lax.axis_index("s")
        w = c * 16 + s
        base = w * cpt
        pltpu.sync_copy(seg_hbm.at[pl.ds(base, cpt)], idx_v)

        def zrow(i, carry):
            zero_v[i, :] = jnp.zeros((16,), F32)
            return carry

        lax.fori_loop(0, _CHUNK, zrow, 0)

        def zchunk(i, carry):
            pltpu.sync_copy(
                zero_v, acc_sh.at[pl.ds(s * nps + i * _CHUNK, _CHUNK)]
            )
            return carry

        lax.fori_loop(0, nps // _CHUNK, zchunk, 0)
        plsc.subcore_barrier()

        def round_body(r, carry):
            pltpu.sync_copy(
                feat_hbm.at[pl.ds((base + r * _ROUND) * _CHUNK, _ROUND * _CHUNK)],
                rows_v,
            )
            for b in range(_ROUND):
                pltpu.async_copy(
                    rows_v.at[pl.ds(b * _CHUNK, _CHUNK)],
                    acc_sh.at[idx_v.at[r * _ROUND + b]],
                    ssem,
                    add=True,
                )
            for b in range(_ROUND):
                pltpu.make_async_copy(
                    rows_v.at[pl.ds(b * _CHUNK, _CHUNK)],
                    acc_sh.at[idx_v.at[r * _ROUND + b]],
                    ssem,
                ).wait()
            return carry

        lax.fori_loop(0, nr, round_body, 0)
        plsc.subcore_barrier()

        pltpu.sync_copy(
            acc_sh.at[pl.ds(s * nps, nps)],
            out_hbm.at[pl.ds(c * n + s * nps, nps)],
        )

    return _scatter(seg2d, feat)


def _fused_body(nodes_ref, kg_ref, seg0_ref, seg1_ref, wq3_ref, bq3_ref,
                tt_ref, w1n_ref, w1a_ref, w1s_ref, b1e_ref, w2_ref, b2_ref,
                g_ref, beta_ref, out_ref):
    x = nodes_ref[...]
    kg = kg_ref[...]
    qw = jnp.dot(x, wq3_ref[...], preferred_element_type=F32) + bq3_ref[...]
    sc = kg * qw
    m = jnp.max(sc, axis=1, keepdims=True)
    e = jnp.exp(sc - m)
    den = jnp.dot(e, tt_ref[...], preferred_element_type=F32)
    num = jnp.dot(e * kg, tt_ref[...], preferred_element_type=F32)
    att = num / den
    seg = seg0_ref[...] + seg1_ref[...]
    h = (
        jnp.dot(x, w1n_ref[...], preferred_element_type=F32)
        + jnp.dot(att, w1a_ref[...], preferred_element_type=F32)
        + jnp.dot(seg, w1s_ref[...], preferred_element_type=F32)
        + b1e_ref[...]
    )
    h = jnp.maximum(h, 0.0)
    h2 = jax.nn.sigmoid(
        jnp.dot(h, w2_ref[...], preferred_element_type=F32) + b2_ref[...]
    )
    mean = jnp.mean(h2, axis=1, keepdims=True)
    ctr = h2 - mean
    var = jnp.mean(ctr * ctr, axis=1, keepdims=True)
    out_ref[...] = ctr / jnp.sqrt(var + 1e-3) * g_ref[...] + beta_ref[...]


def _fused_mlp(nodes, kg, segparts, wq3, bq3, tt, w1n, w1a, w1s, b1e, w2, b2,
               gamma, beta, block_n=1000):
    n = nodes.shape[0]
    assert n % block_n == 0
    nb = n // block_n
    full = lambda a: pl.BlockSpec(a.shape, lambda i: tuple(0 for _ in a.shape))
    return pl.pallas_call(
        _fused_body,
        grid=(nb,),
        in_specs=[
            pl.BlockSpec((block_n, 128), lambda i: (i, 0)),
            pl.BlockSpec((block_n, 128), lambda i: (i, 0)),
            pl.BlockSpec((block_n, 16), lambda i: (i, 0)),
            pl.BlockSpec((block_n, 16), lambda i: (nb + i, 0)),
            full(wq3), full(bq3), full(tt), full(w1n), full(w1a),
            full(w1s), full(b1e), full(w2), full(b2), full(gamma), full(beta),
        ],
        out_specs=pl.BlockSpec((block_n, 128), lambda i: (i, 0)),
        out_shape=jax.ShapeDtypeStruct((n, 128), F32),
    )(nodes, kg, segparts, segparts, wq3, bq3, tt, w1n, w1a, w1s, b1e, w2, b2,
      gamma, beta)


def kernel(nodes, globals_, edges, hyper_feat, edge_ind, seg_ind, W_q, b_q,
           W_k, b_k, W_conv, b_conv, W1, b1, W2, b2, ln_gamma, ln_beta):
    n, dn = nodes.shape
    e, de = edges.shape
    k = edge_ind.shape[1]
    heads = W_conv.shape[2]

    wc = W_conv[1]
    tmat = jnp.tile(jnp.eye(heads, dtype=F32), (1, k))
    wq3 = (W_q @ wc) @ tmat
    bq3 = ((b_q @ wc + b_conv) @ tmat)[None, :]
    wk2 = W_k @ wc
    bk2 = (b_k @ wc + b_conv)[None, :]
    tt = tmat.T

    w1n = W1[:dn]
    w1g = W1[dn:dn + globals_.shape[1]]
    w1a = W1[dn + globals_.shape[1]:dn + globals_.shape[1] + heads]
    w1s = W1[dn + globals_.shape[1] + heads:]
    b1e = (b1 + globals_[0] @ w1g)[None, :]

    kmh = _edge_keys(edges, wk2, bk2)

    idx2d = edge_ind.reshape(n * k // _CHUNK, _CHUNK)
    kg = _gather_rows(idx2d, kmh).reshape(n, k * heads)

    seg2d = seg_ind.reshape(e // _CHUNK, _CHUNK)
    segparts = _segment_sum_parts(seg2d, hyper_feat, n)

    return _fused_mlp(nodes, kg, segparts, wq3, bq3, tt, w1n, w1a, w1s, b1e,
                      W2, b2[None, :], ln_gamma[None, :], ln_beta[None, :])


# speedup vs baseline: 4.7228x; 4.7228x over previous
import functools

import jax
import jax.numpy as jnp
from jax import lax
from jax.experimental import pallas as pl
from jax.experimental.pallas import tpu as pltpu
from jax.experimental.pallas import tpu_sc as plsc

F32 = jnp.float32


def _edgek_body(edges_ref, w_ref, b_ref, out_ref):
    out_ref[...] = (
        jnp.dot(edges_ref[...], w_ref[...], preferred_element_type=F32)
        + b_ref[...]
    )


def _edge_keys(edges, wk2, bk2, block_e=8000):
    e = edges.shape[0]
    assert e % block_e == 0
    return pl.pallas_call(
        _edgek_body,
        grid=(e // block_e,),
        in_specs=[
            pl.BlockSpec((block_e, edges.shape[1]), lambda i: (i, 0)),
            pl.BlockSpec(wk2.shape, lambda i: (0, 0)),
            pl.BlockSpec((1, bk2.shape[1]), lambda i: (0, 0)),
        ],
        out_specs=pl.BlockSpec((block_e, 8), lambda i: (i, 0)),
        out_shape=jax.ShapeDtypeStruct((e, 8), F32),
    )(edges, wk2, bk2)


_NW = 32
_CHUNK = 125
_ROUND = 40
_NPAD = 51200


def _gather_rows(idx2d, table):
    nch = idx2d.shape[0]
    g = nch * _CHUNK
    cpt = nch // _NW
    nr = cpt // _ROUND
    assert cpt * _NW == nch and nr * _ROUND == cpt
    mesh = plsc.VectorSubcoreMesh(core_axis_name="c", subcore_axis_name="s")

    @functools.partial(
        pl.kernel,
        out_type=jax.ShapeDtypeStruct((g, 8), F32),
        mesh=mesh,
        scratch_types=[
            pltpu.VMEM((cpt, _CHUNK), jnp.int32),
            pltpu.VMEM((_ROUND * _CHUNK, 8), F32),
            pltpu.SemaphoreType.DMA,
        ],
        compiler_params=pltpu.CompilerParams(use_tc_tiling_on_sc=False),
    )
    def _gather(idx_hbm, table_hbm, out_hbm, idx_v, rows_v, gsem):
        c = lax.axis_index("c")
        s = lax.axis_index("s")
        w = s * 2 + c
        base = w * cpt
        pltpu.sync_copy(idx_hbm.at[pl.ds(base, cpt)], idx_v)

        def round_body(r, carry):
            for b in range(_ROUND):
                pltpu.async_copy(
                    table_hbm.at[idx_v.at[r * _ROUND + b]],
                    rows_v.at[pl.ds(b * _CHUNK, _CHUNK)],
                    gsem,
                )
            for b in range(_ROUND):
                pltpu.make_async_copy(
                    table_hbm.at[idx_v.at[r * _ROUND + b]],
                    rows_v.at[pl.ds(b * _CHUNK, _CHUNK)],
                    gsem,
                ).wait()
            pltpu.sync_copy(
                rows_v,
                out_hbm.at[pl.ds((base + r * _ROUND) * _CHUNK, _ROUND * _CHUNK)],
            )
            return carry

        lax.fori_loop(0, nr, round_body, 0)

    return _gather(idx2d, table)


def _segment_sum_parts(seg2d, feat, n):
    nch = seg2d.shape[0]
    d = feat.shape[1]
    cpt = nch // _NW
    rs = 20
    nr = cpt // rs
    assert cpt * _NW == nch and nr * rs == cpt
    nps = _NPAD // 16
    zc = 128
    assert n <= _NPAD and nps % zc == 0
    mesh = plsc.VectorSubcoreMesh(core_axis_name="c", subcore_axis_name="s")

    @functools.partial(
        pl.kernel,
        out_type=jax.ShapeDtypeStruct((2, _NPAD, d), F32),
        mesh=mesh,
        scratch_types=[
            pltpu.VMEM((cpt, _CHUNK), jnp.int32),
            pltpu.VMEM((rs * _CHUNK, d), F32),
            pltpu.VMEM((zc, d), F32),
            pltpu.VMEM_SHARED((_NPAD, d), F32),
            pltpu.SemaphoreType.DMA,
        ],
        compiler_params=pltpu.CompilerParams(use_tc_tiling_on_sc=False),
    )
    def _scatter(seg_hbm, feat_hbm, out_hbm, idx_v, rows_v, zero_v, acc_sh, ssem):
        c = lax.axis_index("c")
        s = lax.axis_index("s")
        w = c * 16 + s
        base = w * cpt
        pltpu.sync_copy(seg_hbm.at[pl.ds(base, cpt)], idx_v)

        def zrow(i, carry):
            zero_v[i, :] = jnp.zeros((16,), F32)
            return carry

        lax.fori_loop(0, zc, zrow, 0)

        def zchunk(i, carry):
            pltpu.sync_copy(
                zero_v, acc_sh.at[pl.ds(s * nps + i * zc, zc)]
            )
            return carry

        lax.fori_loop(0, nps // zc, zchunk, 0)
        plsc.subcore_barrier()

        def round_body(r, carry):
            pltpu.sync_copy(
                feat_hbm.at[pl.ds((base + r * rs) * _CHUNK, rs * _CHUNK)],
                rows_v,
            )
            for b in range(rs):
                pltpu.async_copy(
                    rows_v.at[pl.ds(b * _CHUNK, _CHUNK)],
                    acc_sh.at[idx_v.at[r * rs + b]],
                    ssem,
                    add=True,
                )
            for b in range(rs):
                pltpu.make_async_copy(
                    rows_v.at[pl.ds(b * _CHUNK, _CHUNK)],
                    acc_sh.at[idx_v.at[r * rs + b]],
                    ssem,
                ).wait()
            return carry

        lax.fori_loop(0, nr, round_body, 0)
        plsc.subcore_barrier()

        pltpu.sync_copy(
            acc_sh.at[pl.ds(s * nps, nps)],
            out_hbm.at[c, pl.ds(s * nps, nps)],
        )

    return _scatter(seg2d, feat)


def _fused_body(nodes_ref, kg_ref, seg0_ref, seg1_ref, wq3_ref, bq3_ref,
                tt_ref, w1n_ref, w1a_ref, w1s_ref, b1e_ref, w2_ref, b2_ref,
                g_ref, beta_ref, out_ref):
    x = nodes_ref[...]
    kg = kg_ref[...]
    qw = jnp.dot(x, wq3_ref[...], preferred_element_type=F32) + bq3_ref[...]
    sc = kg * qw
    m = jnp.max(sc, axis=1, keepdims=True)
    e = jnp.exp(sc - m)
    den = jnp.dot(e, tt_ref[...], preferred_element_type=F32)
    num = jnp.dot(e * kg, tt_ref[...], preferred_element_type=F32)
    att = num / den
    seg = seg0_ref[0] + seg1_ref[0]
    h = (
        jnp.dot(x, w1n_ref[...], preferred_element_type=F32)
        + jnp.dot(att, w1a_ref[...], preferred_element_type=F32)
        + jnp.dot(seg, w1s_ref[...], preferred_element_type=F32)
        + b1e_ref[...]
    )
    h = jnp.maximum(h, 0.0)
    h2 = jax.nn.sigmoid(
        jnp.dot(h, w2_ref[...], preferred_element_type=F32) + b2_ref[...]
    )
    mean = jnp.mean(h2, axis=1, keepdims=True)
    ctr = h2 - mean
    var = jnp.mean(ctr * ctr, axis=1, keepdims=True)
    out_ref[...] = ctr / jnp.sqrt(var + 1e-3) * g_ref[...] + beta_ref[...]


def _fused_mlp(nodes, kg, segparts, wq3, bq3, tt, w1n, w1a, w1s, b1e, w2, b2,
               gamma, beta, block_n=1000):
    n = nodes.shape[0]
    assert n % block_n == 0
    nb = n // block_n
    full = lambda a: pl.BlockSpec(a.shape, lambda i: tuple(0 for _ in a.shape))
    return pl.pallas_call(
        _fused_body,
        grid=(nb,),
        in_specs=[
            pl.BlockSpec((block_n, 128), lambda i: (i, 0)),
            pl.BlockSpec((block_n, 128), lambda i: (i, 0)),
            pl.BlockSpec((1, block_n, 16), lambda i: (0, i, 0)),
            pl.BlockSpec((1, block_n, 16), lambda i: (1, i, 0)),
            full(wq3), full(bq3), full(tt), full(w1n), full(w1a),
            full(w1s), full(b1e), full(w2), full(b2), full(gamma), full(beta),
        ],
        out_specs=pl.BlockSpec((block_n, 128), lambda i: (i, 0)),
        out_shape=jax.ShapeDtypeStruct((n, 128), F32),
    )(nodes, kg, segparts, segparts, wq3, bq3, tt, w1n, w1a, w1s, b1e, w2, b2,
      gamma, beta)


def kernel(nodes, globals_, edges, hyper_feat, edge_ind, seg_ind, W_q, b_q,
           W_k, b_k, W_conv, b_conv, W1, b1, W2, b2, ln_gamma, ln_beta):
    n, dn = nodes.shape
    e, de = edges.shape
    k = edge_ind.shape[1]
    heads = W_conv.shape[2]

    wc = W_conv[1]
    tmat = jnp.tile(jnp.eye(heads, dtype=F32), (1, k))
    wq3 = (W_q @ wc) @ tmat
    bq3 = ((b_q @ wc + b_conv) @ tmat)[None, :]
    wk2 = W_k @ wc
    bk2 = (b_k @ wc + b_conv)[None, :]
    tt = tmat.T

    w1n = W1[:dn]
    w1g = W1[dn:dn + globals_.shape[1]]
    w1a = W1[dn + globals_.shape[1]:dn + globals_.shape[1] + heads]
    w1s = W1[dn + globals_.shape[1] + heads:]
    b1e = (b1 + globals_[0] @ w1g)[None, :]

    kmh = _edge_keys(edges, wk2, bk2)

    idx2d = edge_ind.reshape(n * k // _CHUNK, _CHUNK)
    kg = _gather_rows(idx2d, kmh).reshape(n, k * heads)

    seg2d = seg_ind.reshape(e // _CHUNK, _CHUNK)
    segparts = _segment_sum_parts(seg2d, hyper_feat, n)

    return _fused_mlp(nodes, kg, segparts, wq3, bq3, tt, w1n, w1a, w1s, b1e,
                      W2, b2[None, :], ln_gamma[None, :], ln_beta[None, :])


# speedup vs baseline: 6.9260x; 1.4665x over previous
import functools

import jax
import jax.numpy as jnp
from jax import lax
from jax.experimental import pallas as pl
from jax.experimental.pallas import tpu as pltpu
from jax.experimental.pallas import tpu_sc as plsc

F32 = jnp.float32


def _edgek_body(edges_ref, w_ref, b_ref, out_ref):
    out_ref[...] = (
        jnp.dot(edges_ref[...], w_ref[...], preferred_element_type=F32)
        + b_ref[...]
    )


def _edge_keys(e256, wbig, bbig, block_r=2000):
    r = e256.shape[0]
    assert r % block_r == 0
    return pl.pallas_call(
        _edgek_body,
        grid=(r // block_r,),
        in_specs=[
            pl.BlockSpec((block_r, 256), lambda i: (i, 0)),
            pl.BlockSpec(wbig.shape, lambda i: (0, 0)),
            pl.BlockSpec((1, bbig.shape[1]), lambda i: (0, 0)),
        ],
        out_specs=pl.BlockSpec((block_r, 128), lambda i: (i, 0)),
        out_shape=jax.ShapeDtypeStruct((r, 128), F32),
    )(e256, wbig, bbig)


_NW = 32
_CHUNK = 125
_ROUND = 40
_NPAD = 51200


def _gather_rows(idx2d, table):
    nch = idx2d.shape[0]
    g = nch * _CHUNK
    cpt = nch // _NW
    nr = cpt // _ROUND
    assert cpt * _NW == nch and nr * _ROUND == cpt
    mesh = plsc.VectorSubcoreMesh(core_axis_name="c", subcore_axis_name="s")

    @functools.partial(
        pl.kernel,
        out_type=jax.ShapeDtypeStruct((g, 8), F32),
        mesh=mesh,
        scratch_types=[
            pltpu.VMEM((cpt, _CHUNK), jnp.int32),
            pltpu.VMEM((_ROUND * _CHUNK, 8), F32),
            pltpu.SemaphoreType.DMA,
        ],
        compiler_params=pltpu.CompilerParams(use_tc_tiling_on_sc=False),
    )
    def _gather(idx_hbm, table_hbm, out_hbm, idx_v, rows_v, gsem):
        c = lax.axis_index("c")
        s = lax.axis_index("s")
        w = s * 2 + c
        base = w * cpt
        pltpu.sync_copy(idx_hbm.at[pl.ds(base, cpt)], idx_v)

        def round_body(r, carry):
            for b in range(_ROUND):
                pltpu.async_copy(
                    table_hbm.at[idx_v.at[r * _ROUND + b]],
                    rows_v.at[pl.ds(b * _CHUNK, _CHUNK)],
                    gsem,
                )
            for b in range(_ROUND):
                pltpu.make_async_copy(
                    table_hbm.at[idx_v.at[r * _ROUND + b]],
                    rows_v.at[pl.ds(b * _CHUNK, _CHUNK)],
                    gsem,
                ).wait()
            pltpu.sync_copy(
                rows_v,
                out_hbm.at[pl.ds((base + r * _ROUND) * _CHUNK, _ROUND * _CHUNK)],
            )
            return carry

        lax.fori_loop(0, nr, round_body, 0)

    return _gather(idx2d, table)


def _segment_sum_parts(seg2d, feat, n):
    nch = seg2d.shape[0]
    d = feat.shape[1]
    cpt = nch // _NW
    rs = 20
    nr = cpt // rs
    assert cpt * _NW == nch and nr * rs == cpt
    nps = _NPAD // 16
    zc = 128
    assert n <= _NPAD and nps % zc == 0
    mesh = plsc.VectorSubcoreMesh(core_axis_name="c", subcore_axis_name="s")

    @functools.partial(
        pl.kernel,
        out_type=jax.ShapeDtypeStruct((2, _NPAD, d), F32),
        mesh=mesh,
        scratch_types=[
            pltpu.VMEM((cpt, _CHUNK), jnp.int32),
            pltpu.VMEM((rs * _CHUNK, d), F32),
            pltpu.VMEM((zc, d), F32),
            pltpu.VMEM_SHARED((_NPAD, d), F32),
            pltpu.SemaphoreType.DMA,
        ],
        compiler_params=pltpu.CompilerParams(use_tc_tiling_on_sc=False),
    )
    def _scatter(seg_hbm, feat_hbm, out_hbm, idx_v, rows_v, zero_v, acc_sh, ssem):
        c = lax.axis_index("c")
        s = lax.axis_index("s")
        w = c * 16 + s
        base = w * cpt
        pltpu.sync_copy(seg_hbm.at[pl.ds(base, cpt)], idx_v)

        def zrow(i, carry):
            zero_v[i, :] = jnp.zeros((16,), F32)
            return carry

        lax.fori_loop(0, zc, zrow, 0)

        def zchunk(i, carry):
            pltpu.sync_copy(
                zero_v, acc_sh.at[pl.ds(s * nps + i * zc, zc)]
            )
            return carry

        lax.fori_loop(0, nps // zc, zchunk, 0)
        plsc.subcore_barrier()

        def round_body(r, carry):
            pltpu.sync_copy(
                feat_hbm.at[pl.ds((base + r * rs) * _CHUNK, rs * _CHUNK)],
                rows_v,
            )
            for b in range(rs):
                pltpu.async_copy(
                    rows_v.at[pl.ds(b * _CHUNK, _CHUNK)],
                    acc_sh.at[idx_v.at[r * rs + b]],
                    ssem,
                    add=True,
                )
            for b in range(rs):
                pltpu.make_async_copy(
                    rows_v.at[pl.ds(b * _CHUNK, _CHUNK)],
                    acc_sh.at[idx_v.at[r * rs + b]],
                    ssem,
                ).wait()
            return carry

        lax.fori_loop(0, nr, round_body, 0)
        plsc.subcore_barrier()

        pltpu.sync_copy(
            acc_sh.at[pl.ds(s * nps, nps)],
            out_hbm.at[c, pl.ds(s * nps, nps)],
        )

    return _scatter(seg2d, feat)


def _fused_body(nodes_ref, kg_ref, seg0_ref, seg1_ref, wq3_ref, bq3_ref,
                tt_ref, w1n_ref, w1a_ref, w1s_ref, b1e_ref, w2_ref, b2_ref,
                g_ref, beta_ref, out_ref):
    x = nodes_ref[...]
    kg = kg_ref[...]
    qw = jnp.dot(x, wq3_ref[...], preferred_element_type=F32) + bq3_ref[...]
    sc = kg * qw
    m = jnp.max(sc, axis=1, keepdims=True)
    e = jnp.exp(sc - m)
    den = jnp.dot(e, tt_ref[...], preferred_element_type=F32)
    num = jnp.dot(e * kg, tt_ref[...], preferred_element_type=F32)
    att = num / den
    seg = seg0_ref[0] + seg1_ref[0]
    h = (
        jnp.dot(x, w1n_ref[...], preferred_element_type=F32)
        + jnp.dot(att, w1a_ref[...], preferred_element_type=F32)
        + jnp.dot(seg, w1s_ref[...], preferred_element_type=F32)
        + b1e_ref[...]
    )
    h = jnp.maximum(h, 0.0)
    h2 = jax.nn.sigmoid(
        jnp.dot(h, w2_ref[...], preferred_element_type=F32) + b2_ref[...]
    )
    mean = jnp.mean(h2, axis=1, keepdims=True)
    ctr = h2 - mean
    var = jnp.mean(ctr * ctr, axis=1, keepdims=True)
    out_ref[...] = ctr / jnp.sqrt(var + 1e-3) * g_ref[...] + beta_ref[...]


def _fused_mlp(nodes, kg, segparts, wq3, bq3, tt, w1n, w1a, w1s, b1e, w2, b2,
               gamma, beta, block_n=1000):
    n = nodes.shape[0]
    assert n % block_n == 0
    nb = n // block_n
    full = lambda a: pl.BlockSpec(a.shape, lambda i: tuple(0 for _ in a.shape))
    return pl.pallas_call(
        _fused_body,
        grid=(nb,),
        in_specs=[
            pl.BlockSpec((block_n, 128), lambda i: (i, 0)),
            pl.BlockSpec((block_n, 128), lambda i: (i, 0)),
            pl.BlockSpec((1, block_n, 16), lambda i: (0, i, 0)),
            pl.BlockSpec((1, block_n, 16), lambda i: (1, i, 0)),
            full(wq3), full(bq3), full(tt), full(w1n), full(w1a),
            full(w1s), full(b1e), full(w2), full(b2), full(gamma), full(beta),
        ],
        out_specs=pl.BlockSpec((block_n, 128), lambda i: (i, 0)),
        out_shape=jax.ShapeDtypeStruct((n, 128), F32),
    )(nodes, kg, segparts, segparts, wq3, bq3, tt, w1n, w1a, w1s, b1e, w2, b2,
      gamma, beta)


def kernel(nodes, globals_, edges, hyper_feat, edge_ind, seg_ind, W_q, b_q,
           W_k, b_k, W_conv, b_conv, W1, b1, W2, b2, ln_gamma, ln_beta):
    n, dn = nodes.shape
    e, de = edges.shape
    k = edge_ind.shape[1]
    heads = W_conv.shape[2]

    wc = W_conv[1]
    tmat = jnp.tile(jnp.eye(heads, dtype=F32), (1, k))
    wq3 = (W_q @ wc) @ tmat
    bq3 = ((b_q @ wc + b_conv) @ tmat)[None, :]
    wk2 = W_k @ wc
    wbig = jnp.kron(jnp.eye(16, dtype=F32), wk2)
    bbig = ((b_k @ wc + b_conv) @ tmat)[None, :]
    tt = tmat.T

    w1n = W1[:dn]
    w1g = W1[dn:dn + globals_.shape[1]]
    w1a = W1[dn + globals_.shape[1]:dn + globals_.shape[1] + heads]
    w1s = W1[dn + globals_.shape[1] + heads:]
    b1e = (b1 + globals_[0] @ w1g)[None, :]

    e256 = edges.reshape(e // 16, 16 * de)
    kmh128 = _edge_keys(e256, wbig, bbig)
    table = kmh128.reshape(e, 8)

    idx2d = edge_ind.reshape(n * k // _CHUNK, _CHUNK)
    kg = _gather_rows(idx2d, table).reshape(n, k * heads)

    hf128 = lax.optimization_barrier(hyper_feat.reshape(e * de // 128, 128))
    seg2d = seg_ind.reshape(e // _CHUNK, _CHUNK)
    segparts = _segment_sum_parts(seg2d, hf128.reshape(e, de), n)

    return _fused_mlp(nodes, kg, segparts, wq3, bq3, tt, w1n, w1a, w1s, b1e,
                      W2, b2[None, :], ln_gamma[None, :], ln_beta[None, :])
